# Initial kernel scaffold; baseline (speedup 1.0000x reference)
import jax
import jax.numpy as jnp
from jax.experimental import pallas as pl


def kernel(y_pred, length, event):
    raise NotImplementedError("write your pallas kernel here")


# speedup vs baseline: 12.8419x; 12.8419x over previous
import jax
import jax.numpy as jnp
from jax import lax
from jax.experimental import pallas as pl
from jax.experimental.pallas import tpu as pltpu

_TOP_N = 2
_REG_W = 0.05
_RBLK = 64
_KEY_LO = 1234


def _threefry_bits(idx_u32):
    ks0 = 0
    ks1 = _KEY_LO
    ks2 = ks0 ^ ks1 ^ 0x1BD11BDA
    rot_a = (13, 15, 26, 6)
    rot_b = (17, 29, 16, 24)

    def rounds(x0, x1, rots):
        for r in rots:
            x0 = x0 + x1
            x1 = ((x1 << r) | (x1 >> (32 - r))) ^ x0
        return x0, x1

    def u32(v):
        return jnp.uint32(v & 0xFFFFFFFF)

    x0 = jnp.zeros_like(idx_u32) + u32(ks0)
    x1 = idx_u32 + u32(ks1)
    x0, x1 = rounds(x0, x1, rot_a)
    x0 = x0 + u32(ks1)
    x1 = x1 + u32(ks2 + 1)
    x0, x1 = rounds(x0, x1, rot_b)
    x0 = x0 + u32(ks2)
    x1 = x1 + u32(ks0 + 2)
    x0, x1 = rounds(x0, x1, rot_a)
    x0 = x0 + u32(ks0)
    x1 = x1 + u32(ks1 + 3)
    x0, x1 = rounds(x0, x1, rot_b)
    x0 = x0 + u32(ks1)
    x1 = x1 + u32(ks2 + 4)
    x0, x1 = rounds(x0, x1, rot_a)
    x0 = x0 + u32(ks2)
    x1 = x1 + u32(ks0 + 5)
    return x0 ^ x1


def _body(yr_ref, lr_ref, yc_ref, lc_ref, ec_ref, out_ref,
          colsum, validv, lossacc):
    n = yr_ref.shape[1]
    k = pl.program_id(0)
    nsteps = pl.num_programs(0)
    r0 = k * _RBLK

    @pl.when(k == 0)
    def _init():
        colsum[...] = jnp.zeros_like(colsum)
        validv[...] = jnp.zeros_like(validv)
        lossacc[...] = jnp.zeros_like(lossacc)

    y_row = yr_ref[...]
    max_y = jnp.max(y_row)
    e_row = jnp.exp(y_row - max_y)
    l_row = lr_ref[...]
    y_c = yc_ref[pl.ds(r0, _RBLK), :]
    l_c = lc_ref[pl.ds(r0, _RBLK), :]
    e_c = ec_ref[pl.ds(r0, _RBLK), :]

    cols = lax.broadcasted_iota(jnp.int32, (_RBLK, n), 1)
    rows = lax.broadcasted_iota(jnp.int32, (_RBLK, n), 0) + r0
    idx = lax.bitcast_convert_type(rows * n + cols, jnp.uint32)
    bits = _threefry_bits(idx)
    fl = lax.bitcast_convert_type((bits >> 9) | jnp.uint32(0x3F800000),
                                  jnp.float32)
    u = fl - 1.0
    pair = ((l_row - l_c) > 0.0) & (e_c > 0.0)
    val = jnp.where(pair, 1.0 + u, 0.0)

    def extract(v):
        m = jnp.max(v, axis=1, keepdims=True)
        ism = v == m
        pos = jnp.min(jnp.where(ism, cols, n), axis=1, keepdims=True)
        sel = cols == pos
        ev = jnp.sum(jnp.where(sel, e_row, 0.0), axis=1, keepdims=True)
        return m, pos, ev, jnp.where(sel, -1.0, v)

    v1, j1, e1, val = extract(val)
    v2, j2, e2, val = extract(val)
    v3 = jnp.max(val, axis=1, keepdims=True)

    s1 = (v1 > v3).astype(jnp.float32)
    s2 = (v2 > v3).astype(jnp.float32)
    validf = s1

    row_sum = s1 * e1 + s2 * e2 + validf * jnp.exp(y_c - max_y)
    rs_safe = jnp.where(validf > 0.0, row_sum, 1.0)
    row_loss = validf * ((max_y - y_c) + jnp.log(rs_safe))
    lossacc[...] += jnp.sum(row_loss, keepdims=True)[:1, :1]

    add = (jnp.where(cols == j1, s1, 0.0) + jnp.where(cols == j2, s2, 0.0))
    colsum[...] += jnp.sum(add, axis=0, keepdims=True)
    validv[pl.ds(r0, _RBLK), :] = validf

    @pl.when(k == nsteps - 1)
    def _finish():
        reg = (jnp.sum(colsum[...] * jnp.abs(y_row), keepdims=True)
               + jnp.sum(validv[...] * jnp.abs(yc_ref[...]), keepdims=True)[:1, :1])
        out_ref[...] = lossacc[...] + _REG_W * reg


def _build_call(n, interpret=False):
    full_row = pl.BlockSpec((1, n), lambda k: (0, 0))
    full_col = pl.BlockSpec((n, 1), lambda k: (0, 0))
    return pl.pallas_call(
        _body,
        grid=(n // _RBLK,),
        in_specs=[full_row, full_row, full_col, full_col, full_col],
        out_specs=pl.BlockSpec((1, 1), lambda k: (0, 0)),
        out_shape=jax.ShapeDtypeStruct((1, 1), jnp.float32),
        scratch_shapes=[
            pltpu.VMEM((1, n), jnp.float32),
            pltpu.VMEM((n, 1), jnp.float32),
            pltpu.VMEM((1, 1), jnp.float32),
        ],
        compiler_params=pltpu.CompilerParams(
            dimension_semantics=("arbitrary",)),
        interpret=interpret,
    )


def kernel(y_pred, length, event):
    n = y_pred.shape[0]
    y_row = y_pred.reshape(1, n)
    l_row = length.reshape(1, n)
    y_col = y_pred.reshape(n, 1)
    l_col = length.reshape(n, 1)
    e_col = event.reshape(n, 1)
    out = _build_call(n)(y_row, l_row, y_col, l_col, e_col)
    return out[0, 0]


# speedup vs baseline: 13.0493x; 1.0161x over previous
import jax
import jax.numpy as jnp
from jax import lax
from jax.experimental import pallas as pl
from jax.experimental.pallas import tpu as pltpu

_TOP_N = 2
_REG_W = 0.05
_RBLK = 128
_KEY_LO = 1234


def _threefry_bits(idx_u32):
    ks0 = 0
    ks1 = _KEY_LO
    ks2 = ks0 ^ ks1 ^ 0x1BD11BDA
    rot_a = (13, 15, 26, 6)
    rot_b = (17, 29, 16, 24)

    def rounds(x0, x1, rots):
        for r in rots:
            x0 = x0 + x1
            x1 = ((x1 << r) | (x1 >> (32 - r))) ^ x0
        return x0, x1

    def u32(v):
        return jnp.uint32(v & 0xFFFFFFFF)

    x0 = jnp.zeros_like(idx_u32) + u32(ks0)
    x1 = idx_u32 + u32(ks1)
    x0, x1 = rounds(x0, x1, rot_a)
    x0 = x0 + u32(ks1)
    x1 = x1 + u32(ks2 + 1)
    x0, x1 = rounds(x0, x1, rot_b)
    x0 = x0 + u32(ks2)
    x1 = x1 + u32(ks0 + 2)
    x0, x1 = rounds(x0, x1, rot_a)
    x0 = x0 + u32(ks0)
    x1 = x1 + u32(ks1 + 3)
    x0, x1 = rounds(x0, x1, rot_b)
    x0 = x0 + u32(ks1)
    x1 = x1 + u32(ks2 + 4)
    x0, x1 = rounds(x0, x1, rot_a)
    x0 = x0 + u32(ks2)
    x1 = x1 + u32(ks0 + 5)
    return x0 ^ x1


def _body(yr_ref, lr_ref, yc_ref, lc_ref, ec_ref, out_ref,
          colsum, validv, lossacc):
    n = yr_ref.shape[1]
    k = pl.program_id(0)
    nsteps = pl.num_programs(0)
    r0 = k * _RBLK

    @pl.when(k == 0)
    def _init():
        colsum[...] = jnp.zeros_like(colsum)
        validv[...] = jnp.zeros_like(validv)
        lossacc[...] = jnp.zeros_like(lossacc)

    y_row = yr_ref[...]
    max_y = jnp.max(y_row)
    e_row = jnp.exp(y_row - max_y)
    l_row = lr_ref[...]
    y_c = yc_ref[pl.ds(r0, _RBLK), :]
    l_c = lc_ref[pl.ds(r0, _RBLK), :]
    e_c = ec_ref[pl.ds(r0, _RBLK), :]

    cols = lax.broadcasted_iota(jnp.int32, (_RBLK, n), 1)
    rows = lax.broadcasted_iota(jnp.int32, (_RBLK, n), 0) + r0
    idx = lax.bitcast_convert_type(rows * n + cols, jnp.uint32)
    bits = _threefry_bits(idx)
    fl = lax.bitcast_convert_type((bits >> 9) | jnp.uint32(0x3F800000),
                                  jnp.float32)
    u = fl - 1.0
    pair = ((l_row - l_c) > 0.0) & (e_c > 0.0)
    val = jnp.where(pair, 1.0 + u, 0.0)

    def extract(v):
        m = jnp.max(v, axis=1, keepdims=True)
        ism = v == m
        pos = jnp.min(jnp.where(ism, cols, n), axis=1, keepdims=True)
        sel = cols == pos
        ev = jnp.sum(jnp.where(sel, e_row, 0.0), axis=1, keepdims=True)
        return m, pos, ev, jnp.where(sel, -1.0, v)

    v1, j1, e1, val = extract(val)
    v2, j2, e2, val = extract(val)
    v3 = jnp.max(val, axis=1, keepdims=True)

    s1 = (v1 > v3).astype(jnp.float32)
    s2 = (v2 > v3).astype(jnp.float32)
    validf = s1

    row_sum = s1 * e1 + s2 * e2 + validf * jnp.exp(y_c - max_y)
    rs_safe = jnp.where(validf > 0.0, row_sum, 1.0)
    row_loss = validf * ((max_y - y_c) + jnp.log(rs_safe))
    lossacc[...] += jnp.sum(row_loss, keepdims=True)[:1, :1]

    add = (jnp.where(cols == j1, s1, 0.0) + jnp.where(cols == j2, s2, 0.0))
    colsum[...] += jnp.sum(add, axis=0, keepdims=True)
    validv[pl.ds(r0, _RBLK), :] = validf

    @pl.when(k == nsteps - 1)
    def _finish():
        reg = (jnp.sum(colsum[...] * jnp.abs(y_row), keepdims=True)
               + jnp.sum(validv[...] * jnp.abs(yc_ref[...]), keepdims=True)[:1, :1])
        out_ref[...] = lossacc[...] + _REG_W * reg


def _build_call(n, interpret=False):
    full_row = pl.BlockSpec((1, n), lambda k: (0, 0))
    full_col = pl.BlockSpec((n, 1), lambda k: (0, 0))
    return pl.pallas_call(
        _body,
        grid=(n // _RBLK,),
        in_specs=[full_row, full_row, full_col, full_col, full_col],
        out_specs=pl.BlockSpec((1, 1), lambda k: (0, 0)),
        out_shape=jax.ShapeDtypeStruct((1, 1), jnp.float32),
        scratch_shapes=[
            pltpu.VMEM((1, n), jnp.float32),
            pltpu.VMEM((n, 1), jnp.float32),
            pltpu.VMEM((1, 1), jnp.float32),
        ],
        compiler_params=pltpu.CompilerParams(
            dimension_semantics=("arbitrary",)),
        interpret=interpret,
    )


def kernel(y_pred, length, event):
    n = y_pred.shape[0]
    y_row = y_pred.reshape(1, n)
    l_row = length.reshape(1, n)
    y_col = y_pred.reshape(n, 1)
    l_col = length.reshape(n, 1)
    e_col = event.reshape(n, 1)
    out = _build_call(n)(y_row, l_row, y_col, l_col, e_col)
    return out[0, 0]
